# baseline (speedup 1.0000x reference)
import functools

import jax
import jax.numpy as jnp
from jax import lax
from jax.experimental import pallas as pl
from jax.experimental.pallas import tpu as pltpu
from jax.experimental.pallas import tpu_sc as plsc

VOCAB = 100000
Q_HEADS = 16
LATENT_DIM = 64
BATCH = 64
CONTEXT = 2048

NC = 2
NS = 16
NW = NC * NS
POS_PER_W = CONTEXT // NW
NB = 8
N_BLK = BATCH // NB
NBUF = 3


def _lane_swap(x):
    perm = jax.lax.iota(jnp.int32, 16) ^ 1
    dnums = lax.GatherDimensionNumbers(
        offset_dims=(), collapsed_slice_dims=(0,), start_index_map=(0,))
    return lax.gather(x, perm[:, None], dnums, (1,),
                      mode=lax.GatherScatterMode.PROMISE_IN_BOUNDS)


def _rope_cache():
    pos = jnp.arange(CONTEXT + 1, dtype=jnp.float32)
    inv_freq = 1.0 / (10000.0 ** (
        jnp.arange(0, LATENT_DIM, 2, dtype=jnp.float32) / LATENT_DIM))
    freqs = pos[:, None] * inv_freq[None, :]
    emb = jnp.repeat(freqs, 2, axis=-1)
    alt = jnp.where(jnp.arange(LATENT_DIM) % 2 == 0, -1.0, 1.0).astype(jnp.float32)
    return jnp.cos(emb), jnp.sin(emb) * alt


def _rope_block(rows_v, buf, cos_v, ss_v):
    def r_body(r, carry):
        cs = [cos_v[r, pl.ds(16 * j, 16)] for j in range(4)]
        sg = [ss_v[r, pl.ds(16 * j, 16)] for j in range(4)]

        def b_body(bl, inner):
            for j in range(4):
                x = rows_v[buf, bl, r, pl.ds(16 * j, 16)]
                rows_v[buf, bl, r, pl.ds(16 * j, 16)] = (
                    x * cs[j] + _lane_swap(x) * sg[j])
            return inner

        lax.fori_loop(0, NB, b_body, carry, unroll=4)
        return carry

    lax.fori_loop(0, POS_PER_W, r_body, 0)


def _make_body(b_base, nbatch):
    n_blk = nbatch // NB

    def _body(ctx_hbm, ltab_hbm, cos_hbm, ss_hbm, out_lat,
              cos_v, ss_v, tok_v, rows_v, gsem, ssem):
        wid = lax.axis_index("s") * NC + lax.axis_index("c")
        pos0 = wid * POS_PER_W

        pltpu.sync_copy(cos_hbm.at[pl.ds(pos0, POS_PER_W)], cos_v)
        pltpu.sync_copy(ss_hbm.at[pl.ds(pos0, POS_PER_W)], ss_v)
        tok_copies = [
            pltpu.async_copy(
                ctx_hbm.at[pl.ds((b_base + b) * CONTEXT + pos0, POS_PER_W)],
                tok_v.at[b], gsem)
            for b in range(nbatch)
        ]
        for c in tok_copies:
            c.wait()

        def gather_block(blk, buf):
            return [
                pltpu.async_copy(ltab_hbm.at[tok_v.at[blk * NB + bl]],
                                 rows_v.at[buf].at[bl], gsem)
                for bl in range(NB)
            ]

        def scatter_block(blk, buf):
            return [
                pltpu.async_copy(
                    rows_v.at[buf].at[bl],
                    out_lat.at[blk * NB + bl].at[pl.ds(pos0, POS_PER_W)],
                    ssem)
                for bl in range(NB)
            ]

        g_prev = gather_block(0, 0)
        s_prev = None
        for blk in range(n_blk):
            buf = blk % NBUF
            for c in g_prev:
                c.wait()
            if blk + 1 < n_blk:
                if s_prev is not None:
                    for c in s_prev:
                        c.wait()
                g_prev = gather_block(blk + 1, (blk + 1) % NBUF)
            _rope_block(rows_v, buf, cos_v, ss_v)
            if s_prev is not None and blk + 1 >= n_blk:
                for c in s_prev:
                    c.wait()
            s_prev = scatter_block(blk, buf)
        for c in s_prev:
            c.wait()

    return _body


def _q_body(nt_sref, qtab_ref, cos_ref, ss_ref, out_ref, panels, xs, sem):
    copies = [
        pltpu.make_async_copy(
            qtab_ref.at[pl.ds((nt_sref[i] // 8) * 8, 8), :],
            panels.at[i], sem)
        for i in range(BATCH)
    ]
    for c in copies:
        c.start()
    for c in copies:
        c.wait()
    for i in range(BATCH):
        xs[pl.ds(i, 1), :] = panels[i, pl.ds(nt_sref[i] % 8, 1), :]
    x = xs[...]
    dim = Q_HEADS * LATENT_DIM
    r_i = lax.broadcasted_iota(jnp.int32, (dim, dim), 0)
    c_i = lax.broadcasted_iota(jnp.int32, (dim, dim), 1)
    pss = jnp.where(r_i == (c_i ^ 1), jnp.broadcast_to(ss_ref[...], (dim, dim)),
                    0.0)
    out_ref[...] = x * cos_ref[...] + jnp.dot(
        x, pss, preferred_element_type=jnp.float32)


@jax.jit
def kernel(context_tokens, next_tokens, q_table, latent_table):
    cos_t, ss_t = _rope_cache()
    cos_ctx = cos_t[:CONTEXT]
    ss_ctx = ss_t[:CONTEXT]
    dim = Q_HEADS * LATENT_DIM
    cos_q = jnp.tile(cos_t[CONTEXT], Q_HEADS)[None, :]
    ss_q = jnp.tile(ss_t[CONTEXT], Q_HEADS)[None, :]

    mesh = plsc.VectorSubcoreMesh(core_axis_name="c", subcore_axis_name="s")
    run = functools.partial(
        pl.kernel,
        mesh=mesh,
        compiler_params=pltpu.CompilerParams(use_tc_tiling_on_sc=False),
        out_type=[
            jax.ShapeDtypeStruct((BATCH, CONTEXT, LATENT_DIM), jnp.float32),
        ],
        scratch_types=[
            pltpu.VMEM((POS_PER_W, LATENT_DIM), jnp.float32),
            pltpu.VMEM((POS_PER_W, LATENT_DIM), jnp.float32),
            pltpu.VMEM((BATCH, POS_PER_W), jnp.int32),
            pltpu.VMEM((NBUF, NB, POS_PER_W, LATENT_DIM), jnp.float32),
            pltpu.SemaphoreType.DMA,
            pltpu.SemaphoreType.DMA,
        ],
    )(_make_body(0, BATCH))
    (out_lat,) = run(
        context_tokens.astype(jnp.int32).reshape(-1),
        latent_table, cos_ctx, ss_ctx)

    out_q = pl.pallas_call(
        _q_body,
        grid_spec=pltpu.PrefetchScalarGridSpec(
            num_scalar_prefetch=1,
            grid=(1,),
            in_specs=[
                pl.BlockSpec(memory_space=pl.ANY),
                pl.BlockSpec((1, dim), lambda i, nt: (0, 0)),
                pl.BlockSpec((1, dim), lambda i, nt: (0, 0)),
            ],
            out_specs=pl.BlockSpec((BATCH, dim), lambda i, nt: (0, 0)),
            scratch_shapes=[
                pltpu.VMEM((BATCH, 8, dim), jnp.float32),
                pltpu.VMEM((BATCH, dim), jnp.float32),
                pltpu.SemaphoreType.DMA,
            ],
        ),
        out_shape=jax.ShapeDtypeStruct((BATCH, dim), jnp.float32),
    )(next_tokens.astype(jnp.int32), q_table, cos_q, ss_q)

    q = out_q.reshape(BATCH, Q_HEADS, 1, LATENT_DIM)
    return (q, out_lat)


# speedup vs baseline: 1.0872x; 1.0872x over previous
import functools

import jax
import jax.numpy as jnp
from jax import lax
from jax.experimental import pallas as pl
from jax.experimental.pallas import tpu as pltpu
from jax.experimental.pallas import tpu_sc as plsc

VOCAB = 100000
Q_HEADS = 16
LATENT_DIM = 64
BATCH = 64
CONTEXT = 2048

NC = 2
NS = 16
NW = NC * NS
POS_PER_W = CONTEXT // NW
NB = 8
N_BLK = BATCH // NB
NBUF = 2


def _lane_swap(x):
    perm = jax.lax.iota(jnp.int32, 16) ^ 1
    dnums = lax.GatherDimensionNumbers(
        offset_dims=(), collapsed_slice_dims=(0,), start_index_map=(0,))
    return lax.gather(x, perm[:, None], dnums, (1,),
                      mode=lax.GatherScatterMode.PROMISE_IN_BOUNDS)


def _rope_cache():
    pos = jnp.arange(CONTEXT + 1, dtype=jnp.float32)
    inv_freq = 1.0 / (10000.0 ** (
        jnp.arange(0, LATENT_DIM, 2, dtype=jnp.float32) / LATENT_DIM))
    freqs = pos[:, None] * inv_freq[None, :]
    emb = jnp.repeat(freqs, 2, axis=-1)
    alt = jnp.where(jnp.arange(LATENT_DIM) % 2 == 0, -1.0, 1.0).astype(jnp.float32)
    return jnp.cos(emb), jnp.sin(emb) * alt


def _rope_block(rows_v, buf, cos_v, ss_v):
    def r_body(r, carry):
        cs = [cos_v[r, pl.ds(16 * j, 16)] for j in range(4)]
        sg = [ss_v[r, pl.ds(16 * j, 16)] for j in range(4)]

        def b_body(bl, inner):
            for j in range(4):
                x = rows_v[buf, bl, r, pl.ds(16 * j, 16)]
                rows_v[buf, bl, r, pl.ds(16 * j, 16)] = (
                    x * cs[j] + _lane_swap(x) * sg[j])
            return inner

        lax.fori_loop(0, NB, b_body, carry, unroll=2)
        return carry

    lax.fori_loop(0, POS_PER_W, r_body, 0)


def _make_body(b_base, nbatch):
    n_blk = nbatch // NB

    def _body(ctx_hbm, ltab_hbm, cos_hbm, ss_hbm, out_lat,
              cos_v, ss_v, tok_v, rows_v, gsem, ssem):
        wid = lax.axis_index("s") * NC + lax.axis_index("c")
        pos0 = wid * POS_PER_W

        pltpu.sync_copy(cos_hbm.at[pl.ds(pos0, POS_PER_W)], cos_v)
        pltpu.sync_copy(ss_hbm.at[pl.ds(pos0, POS_PER_W)], ss_v)
        tok_copies = [
            pltpu.async_copy(
                ctx_hbm.at[pl.ds((b_base + b) * CONTEXT + pos0, POS_PER_W)],
                tok_v.at[b], gsem)
            for b in range(nbatch)
        ]
        for c in tok_copies:
            c.wait()

        def gather_block(blk, buf):
            return [
                pltpu.async_copy(ltab_hbm.at[tok_v.at[blk * NB + bl]],
                                 rows_v.at[buf].at[bl], gsem)
                for bl in range(NB)
            ]

        def scatter_block(blk, buf):
            return [
                pltpu.async_copy(
                    rows_v.at[buf].at[bl],
                    out_lat.at[blk * NB + bl].at[pl.ds(pos0, POS_PER_W)],
                    ssem)
                for bl in range(NB)
            ]

        g_prev = gather_block(0, 0)
        s_prev = None
        for blk in range(n_blk):
            buf = blk % NBUF
            for c in g_prev:
                c.wait()
            if blk + 1 < n_blk:
                if s_prev is not None:
                    for c in s_prev:
                        c.wait()
                g_prev = gather_block(blk + 1, (blk + 1) % NBUF)
            _rope_block(rows_v, buf, cos_v, ss_v)
            if s_prev is not None and blk + 1 >= n_blk:
                for c in s_prev:
                    c.wait()
            s_prev = scatter_block(blk, buf)
        for c in s_prev:
            c.wait()

    return _body


def _q_body(nt_sref, qtab_ref, cos_ref, ss_ref, out_ref, panels, xs, sem):
    copies = [
        pltpu.make_async_copy(
            qtab_ref.at[pl.ds((nt_sref[i] // 8) * 8, 8), :],
            panels.at[i], sem)
        for i in range(BATCH)
    ]
    for c in copies:
        c.start()
    for c in copies:
        c.wait()
    for i in range(BATCH):
        xs[pl.ds(i, 1), :] = panels[i, pl.ds(nt_sref[i] % 8, 1), :]
    x = xs[...]
    dim = Q_HEADS * LATENT_DIM
    r_i = lax.broadcasted_iota(jnp.int32, (dim, dim), 0)
    c_i = lax.broadcasted_iota(jnp.int32, (dim, dim), 1)
    pss = jnp.where(r_i == (c_i ^ 1), jnp.broadcast_to(ss_ref[...], (dim, dim)),
                    0.0)
    out_ref[...] = x * cos_ref[...] + jnp.dot(
        x, pss, preferred_element_type=jnp.float32)


@jax.jit
def kernel(context_tokens, next_tokens, q_table, latent_table):
    cos_t, ss_t = _rope_cache()
    cos_ctx = cos_t[:CONTEXT]
    ss_ctx = ss_t[:CONTEXT]
    dim = Q_HEADS * LATENT_DIM
    cos_q = jnp.tile(cos_t[CONTEXT], Q_HEADS)[None, :]
    ss_q = jnp.tile(ss_t[CONTEXT], Q_HEADS)[None, :]

    mesh = plsc.VectorSubcoreMesh(core_axis_name="c", subcore_axis_name="s")
    run = functools.partial(
        pl.kernel,
        mesh=mesh,
        compiler_params=pltpu.CompilerParams(use_tc_tiling_on_sc=False),
        out_type=[
            jax.ShapeDtypeStruct((BATCH, CONTEXT, LATENT_DIM), jnp.float32),
        ],
        scratch_types=[
            pltpu.VMEM((POS_PER_W, LATENT_DIM), jnp.float32),
            pltpu.VMEM((POS_PER_W, LATENT_DIM), jnp.float32),
            pltpu.VMEM((BATCH, POS_PER_W), jnp.int32),
            pltpu.VMEM((NBUF, NB, POS_PER_W, LATENT_DIM), jnp.float32),
            pltpu.SemaphoreType.DMA,
            pltpu.SemaphoreType.DMA,
        ],
    )(_make_body(0, BATCH))
    (out_lat,) = run(
        context_tokens.astype(jnp.int32).reshape(-1),
        latent_table, cos_ctx, ss_ctx)

    out_q = pl.pallas_call(
        _q_body,
        grid_spec=pltpu.PrefetchScalarGridSpec(
            num_scalar_prefetch=1,
            grid=(1,),
            in_specs=[
                pl.BlockSpec(memory_space=pl.ANY),
                pl.BlockSpec((1, dim), lambda i, nt: (0, 0)),
                pl.BlockSpec((1, dim), lambda i, nt: (0, 0)),
            ],
            out_specs=pl.BlockSpec((BATCH, dim), lambda i, nt: (0, 0)),
            scratch_shapes=[
                pltpu.VMEM((BATCH, 8, dim), jnp.float32),
                pltpu.VMEM((BATCH, dim), jnp.float32),
                pltpu.SemaphoreType.DMA,
            ],
        ),
        out_shape=jax.ShapeDtypeStruct((BATCH, dim), jnp.float32),
    )(next_tokens.astype(jnp.int32), q_table, cos_q, ss_q)

    q = out_q.reshape(BATCH, Q_HEADS, 1, LATENT_DIM)
    return (q, out_lat)


# speedup vs baseline: 1.0899x; 1.0025x over previous
import functools

import jax
import jax.numpy as jnp
from jax import lax
from jax.experimental import pallas as pl
from jax.experimental.pallas import tpu as pltpu
from jax.experimental.pallas import tpu_sc as plsc

VOCAB = 100000
Q_HEADS = 16
LATENT_DIM = 64
BATCH = 64
CONTEXT = 2048

NC = 2
NS = 16
NW = NC * NS
POS_PER_W = CONTEXT // NW
NB = 4
N_BLK = BATCH // NB
NBUF = 2


def _lane_swap(x):
    perm = jax.lax.iota(jnp.int32, 16) ^ 1
    dnums = lax.GatherDimensionNumbers(
        offset_dims=(), collapsed_slice_dims=(0,), start_index_map=(0,))
    return lax.gather(x, perm[:, None], dnums, (1,),
                      mode=lax.GatherScatterMode.PROMISE_IN_BOUNDS)


def _rope_cache():
    pos = jnp.arange(CONTEXT + 1, dtype=jnp.float32)
    inv_freq = 1.0 / (10000.0 ** (
        jnp.arange(0, LATENT_DIM, 2, dtype=jnp.float32) / LATENT_DIM))
    freqs = pos[:, None] * inv_freq[None, :]
    emb = jnp.repeat(freqs, 2, axis=-1)
    alt = jnp.where(jnp.arange(LATENT_DIM) % 2 == 0, -1.0, 1.0).astype(jnp.float32)
    return jnp.cos(emb), jnp.sin(emb) * alt


ROW_SZ = POS_PER_W * LATENT_DIM


def _rope_block(rows_v, flat_v, buf, cos_v, ss_v):
    def r_body(r, carry):
        cs = [cos_v[r, pl.ds(16 * j, 16)] for j in range(4)]
        sg = [ss_v[r, pl.ds(16 * j, 16)] for j in range(4)]

        def b_body(bl, inner):
            for j in range(4):
                x = rows_v[buf, bl, r, pl.ds(16 * j, 16)]
                flat_v[buf, pl.ds(bl * ROW_SZ + r * LATENT_DIM + 16 * j, 16)] = (
                    x * cs[j] + _lane_swap(x) * sg[j])
            return inner

        lax.fori_loop(0, NB, b_body, carry, unroll=2)
        return carry

    lax.fori_loop(0, POS_PER_W, r_body, 0)


def _make_body(b_base, nbatch):
    n_blk = nbatch // NB

    def _body(ctx_hbm, ltab_hbm, cos_hbm, ss_hbm, out_lat,
              cos_v, ss_v, tok_v, rows_v, flat_v, gsem, ssem):
        wid = lax.axis_index("s") * NC + lax.axis_index("c")
        pos0 = wid * POS_PER_W

        pltpu.sync_copy(cos_hbm.at[pl.ds(pos0, POS_PER_W)], cos_v)
        pltpu.sync_copy(ss_hbm.at[pl.ds(pos0, POS_PER_W)], ss_v)
        tok_copies = [
            pltpu.async_copy(
                ctx_hbm.at[pl.ds((b_base + b) * CONTEXT + pos0, POS_PER_W)],
                tok_v.at[b], gsem)
            for b in range(nbatch)
        ]
        for c in tok_copies:
            c.wait()

        def gather_block(blk, buf):
            return [
                pltpu.async_copy(ltab_hbm.at[tok_v.at[blk * NB + bl]],
                                 rows_v.at[buf].at[bl], gsem)
                for bl in range(NB)
            ]

        def scatter_block(blk, buf):
            return [
                pltpu.async_copy(
                    flat_v.at[buf].at[pl.ds(bl * ROW_SZ, ROW_SZ)],
                    out_lat.at[pl.ds(
                        (b_base + blk * NB + bl) * CONTEXT * LATENT_DIM
                        + pos0 * LATENT_DIM, ROW_SZ)],
                    ssem)
                for bl in range(NB)
            ]

        g_prev = gather_block(0, 0)
        s_prev = None
        for blk in range(n_blk):
            buf = blk % NBUF
            for c in g_prev:
                c.wait()
            if blk + 1 < n_blk:
                if s_prev is not None:
                    for c in s_prev:
                        c.wait()
                g_prev = gather_block(blk + 1, (blk + 1) % NBUF)
            _rope_block(rows_v, flat_v, buf, cos_v, ss_v)
            if s_prev is not None and blk + 1 >= n_blk:
                for c in s_prev:
                    c.wait()
            s_prev = scatter_block(blk, buf)
        for c in s_prev:
            c.wait()

    return _body


def _q_body(nt_sref, qtab_ref, cos_ref, ss_ref, out_ref, panels, xs, sem):
    copies = [
        pltpu.make_async_copy(
            qtab_ref.at[pl.ds((nt_sref[i] // 8) * 8, 8), :],
            panels.at[i], sem)
        for i in range(BATCH)
    ]
    for c in copies:
        c.start()
    for c in copies:
        c.wait()
    for i in range(BATCH):
        xs[pl.ds(i, 1), :] = panels[i, pl.ds(nt_sref[i] % 8, 1), :]
    x = xs[...]
    dim = Q_HEADS * LATENT_DIM
    r_i = lax.broadcasted_iota(jnp.int32, (dim, dim), 0)
    c_i = lax.broadcasted_iota(jnp.int32, (dim, dim), 1)
    pss = jnp.where(r_i == (c_i ^ 1), jnp.broadcast_to(ss_ref[...], (dim, dim)),
                    0.0)
    out_ref[...] = x * cos_ref[...] + jnp.dot(
        x, pss, preferred_element_type=jnp.float32)


@jax.jit
def kernel(context_tokens, next_tokens, q_table, latent_table):
    cos_t, ss_t = _rope_cache()
    cos_ctx = cos_t[:CONTEXT]
    ss_ctx = ss_t[:CONTEXT]
    dim = Q_HEADS * LATENT_DIM
    cos_q = jnp.tile(cos_t[CONTEXT], Q_HEADS)[None, :]
    ss_q = jnp.tile(ss_t[CONTEXT], Q_HEADS)[None, :]

    mesh = plsc.VectorSubcoreMesh(core_axis_name="c", subcore_axis_name="s")
    run = functools.partial(
        pl.kernel,
        mesh=mesh,
        compiler_params=pltpu.CompilerParams(use_tc_tiling_on_sc=False),
        out_type=[
            jax.ShapeDtypeStruct((BATCH * CONTEXT * LATENT_DIM,), jnp.float32),
        ],
        scratch_types=[
            pltpu.VMEM((POS_PER_W, LATENT_DIM), jnp.float32),
            pltpu.VMEM((POS_PER_W, LATENT_DIM), jnp.float32),
            pltpu.VMEM((BATCH, POS_PER_W), jnp.int32),
            pltpu.VMEM((NBUF, NB, POS_PER_W, LATENT_DIM), jnp.float32),
            pltpu.VMEM((NBUF, NB * POS_PER_W * LATENT_DIM), jnp.float32),
            pltpu.SemaphoreType.DMA,
            pltpu.SemaphoreType.DMA,
        ],
    )(_make_body(0, BATCH))
    (out_flat,) = run(
        context_tokens.astype(jnp.int32).reshape(-1),
        latent_table, cos_ctx, ss_ctx)
    out_lat = out_flat.reshape(BATCH, CONTEXT, LATENT_DIM)

    out_q = pl.pallas_call(
        _q_body,
        grid_spec=pltpu.PrefetchScalarGridSpec(
            num_scalar_prefetch=1,
            grid=(1,),
            in_specs=[
                pl.BlockSpec(memory_space=pl.ANY),
                pl.BlockSpec((1, dim), lambda i, nt: (0, 0)),
                pl.BlockSpec((1, dim), lambda i, nt: (0, 0)),
            ],
            out_specs=pl.BlockSpec((BATCH, dim), lambda i, nt: (0, 0)),
            scratch_shapes=[
                pltpu.VMEM((BATCH, 8, dim), jnp.float32),
                pltpu.VMEM((BATCH, dim), jnp.float32),
                pltpu.SemaphoreType.DMA,
            ],
        ),
        out_shape=jax.ShapeDtypeStruct((BATCH, dim), jnp.float32),
    )(next_tokens.astype(jnp.int32), q_table, cos_q, ss_q)

    q = out_q.reshape(BATCH, Q_HEADS, 1, LATENT_DIM)
    return (q, out_lat)


# speedup vs baseline: 1.0904x; 1.0004x over previous
import functools

import jax
import jax.numpy as jnp
from jax import lax
from jax.experimental import pallas as pl
from jax.experimental.pallas import tpu as pltpu
from jax.experimental.pallas import tpu_sc as plsc

VOCAB = 100000
Q_HEADS = 16
LATENT_DIM = 64
BATCH = 64
CONTEXT = 2048

NC = 2
NS = 16
NW = NC * NS
POS_PER_W = CONTEXT // NW
NB = 4
N_BLK = BATCH // NB
NBUF = 3


def _lane_swap(x):
    perm = jax.lax.iota(jnp.int32, 16) ^ 1
    dnums = lax.GatherDimensionNumbers(
        offset_dims=(), collapsed_slice_dims=(0,), start_index_map=(0,))
    return lax.gather(x, perm[:, None], dnums, (1,),
                      mode=lax.GatherScatterMode.PROMISE_IN_BOUNDS)


def _rope_cache():
    pos = jnp.arange(CONTEXT + 1, dtype=jnp.float32)
    inv_freq = 1.0 / (10000.0 ** (
        jnp.arange(0, LATENT_DIM, 2, dtype=jnp.float32) / LATENT_DIM))
    freqs = pos[:, None] * inv_freq[None, :]
    emb = jnp.repeat(freqs, 2, axis=-1)
    alt = jnp.where(jnp.arange(LATENT_DIM) % 2 == 0, -1.0, 1.0).astype(jnp.float32)
    return jnp.cos(emb), jnp.sin(emb) * alt


ROW_SZ = POS_PER_W * LATENT_DIM


def _rope_block(rows_v, flat_v, buf, cos_v, ss_v):
    def r_body(r, carry):
        cs = [cos_v[r, pl.ds(16 * j, 16)] for j in range(4)]
        sg = [ss_v[r, pl.ds(16 * j, 16)] for j in range(4)]

        def b_body(bl, inner):
            for j in range(4):
                x = rows_v[buf, bl, r, pl.ds(16 * j, 16)]
                flat_v[buf, pl.ds(bl * ROW_SZ + r * LATENT_DIM + 16 * j, 16)] = (
                    x * cs[j] + _lane_swap(x) * sg[j])
            return inner

        lax.fori_loop(0, NB, b_body, carry, unroll=2)
        return carry

    lax.fori_loop(0, POS_PER_W, r_body, 0)


def _make_body(b_base, nbatch):
    n_blk = nbatch // NB

    def _body(ctx_hbm, ltab_hbm, cos_hbm, ss_hbm, out_lat,
              cos_v, ss_v, tok_v, rows_v, flat_v, gsem, ssem):
        wid = lax.axis_index("s") * NC + lax.axis_index("c")
        pos0 = wid * POS_PER_W

        pltpu.sync_copy(cos_hbm.at[pl.ds(pos0, POS_PER_W)], cos_v)
        pltpu.sync_copy(ss_hbm.at[pl.ds(pos0, POS_PER_W)], ss_v)
        tok_copies = [
            pltpu.async_copy(
                ctx_hbm.at[pl.ds((b_base + b) * CONTEXT + pos0, POS_PER_W)],
                tok_v.at[b], gsem)
            for b in range(nbatch)
        ]
        for c in tok_copies:
            c.wait()

        def gather_block(blk, buf):
            return [
                pltpu.async_copy(ltab_hbm.at[tok_v.at[blk * NB + bl]],
                                 rows_v.at[buf].at[bl], gsem)
                for bl in range(NB)
            ]

        def scatter_block(blk, buf):
            return [
                pltpu.async_copy(
                    flat_v.at[buf].at[pl.ds(bl * ROW_SZ, ROW_SZ)],
                    out_lat.at[pl.ds(
                        (b_base + blk * NB + bl) * CONTEXT * LATENT_DIM
                        + pos0 * LATENT_DIM, ROW_SZ)],
                    ssem)
                for bl in range(NB)
            ]

        g_prev = gather_block(0, 0)
        s_prev = None
        for blk in range(n_blk):
            buf = blk % NBUF
            for c in g_prev:
                c.wait()
            if blk + 1 < n_blk:
                if s_prev is not None:
                    for c in s_prev:
                        c.wait()
                g_prev = gather_block(blk + 1, (blk + 1) % NBUF)
            _rope_block(rows_v, flat_v, buf, cos_v, ss_v)
            if s_prev is not None and blk + 1 >= n_blk:
                for c in s_prev:
                    c.wait()
            s_prev = scatter_block(blk, buf)
        for c in s_prev:
            c.wait()

    return _body


def _q_body(nt_sref, qtab_ref, cos_ref, ss_ref, out_ref, panels, xs, sem):
    copies = [
        pltpu.make_async_copy(
            qtab_ref.at[pl.ds((nt_sref[i] // 8) * 8, 8), :],
            panels.at[i], sem)
        for i in range(BATCH)
    ]
    for c in copies:
        c.start()
    for c in copies:
        c.wait()
    for i in range(BATCH):
        xs[pl.ds(i, 1), :] = panels[i, pl.ds(nt_sref[i] % 8, 1), :]
    x = xs[...]
    dim = Q_HEADS * LATENT_DIM
    r_i = lax.broadcasted_iota(jnp.int32, (dim, dim), 0)
    c_i = lax.broadcasted_iota(jnp.int32, (dim, dim), 1)
    pss = jnp.where(r_i == (c_i ^ 1), jnp.broadcast_to(ss_ref[...], (dim, dim)),
                    0.0)
    out_ref[...] = x * cos_ref[...] + jnp.dot(
        x, pss, preferred_element_type=jnp.float32)


@jax.jit
def kernel(context_tokens, next_tokens, q_table, latent_table):
    cos_t, ss_t = _rope_cache()
    cos_ctx = cos_t[:CONTEXT]
    ss_ctx = ss_t[:CONTEXT]
    dim = Q_HEADS * LATENT_DIM
    cos_q = jnp.tile(cos_t[CONTEXT], Q_HEADS)[None, :]
    ss_q = jnp.tile(ss_t[CONTEXT], Q_HEADS)[None, :]

    mesh = plsc.VectorSubcoreMesh(core_axis_name="c", subcore_axis_name="s")
    run = functools.partial(
        pl.kernel,
        mesh=mesh,
        compiler_params=pltpu.CompilerParams(use_tc_tiling_on_sc=False),
        out_type=[
            jax.ShapeDtypeStruct((BATCH * CONTEXT * LATENT_DIM,), jnp.float32),
        ],
        scratch_types=[
            pltpu.VMEM((POS_PER_W, LATENT_DIM), jnp.float32),
            pltpu.VMEM((POS_PER_W, LATENT_DIM), jnp.float32),
            pltpu.VMEM((BATCH, POS_PER_W), jnp.int32),
            pltpu.VMEM((NBUF, NB, POS_PER_W, LATENT_DIM), jnp.float32),
            pltpu.VMEM((NBUF, NB * POS_PER_W * LATENT_DIM), jnp.float32),
            pltpu.SemaphoreType.DMA,
            pltpu.SemaphoreType.DMA,
        ],
    )(_make_body(0, BATCH))
    (out_flat,) = run(
        context_tokens.astype(jnp.int32).reshape(-1),
        latent_table, cos_ctx, ss_ctx)
    out_lat = out_flat.reshape(BATCH, CONTEXT, LATENT_DIM)

    out_q = pl.pallas_call(
        _q_body,
        grid_spec=pltpu.PrefetchScalarGridSpec(
            num_scalar_prefetch=1,
            grid=(1,),
            in_specs=[
                pl.BlockSpec(memory_space=pl.ANY),
                pl.BlockSpec((1, dim), lambda i, nt: (0, 0)),
                pl.BlockSpec((1, dim), lambda i, nt: (0, 0)),
            ],
            out_specs=pl.BlockSpec((BATCH, dim), lambda i, nt: (0, 0)),
            scratch_shapes=[
                pltpu.VMEM((BATCH, 8, dim), jnp.float32),
                pltpu.VMEM((BATCH, dim), jnp.float32),
                pltpu.SemaphoreType.DMA,
            ],
        ),
        out_shape=jax.ShapeDtypeStruct((BATCH, dim), jnp.float32),
    )(next_tokens.astype(jnp.int32), q_table, cos_q, ss_q)

    q = out_q.reshape(BATCH, Q_HEADS, 1, LATENT_DIM)
    return (q, out_lat)


# speedup vs baseline: 1.0924x; 1.0019x over previous
import functools

import jax
import jax.numpy as jnp
from jax import lax
from jax.experimental import pallas as pl
from jax.experimental.pallas import tpu as pltpu
from jax.experimental.pallas import tpu_sc as plsc

VOCAB = 100000
Q_HEADS = 16
LATENT_DIM = 64
BATCH = 64
CONTEXT = 2048

NC = 2
NS = 16
NW = NC * NS
POS_PER_W = CONTEXT // NW
NB = 4
N_BLK = BATCH // NB
NBUF = 2


def _lane_swap(x):
    perm = jax.lax.iota(jnp.int32, 16) ^ 1
    dnums = lax.GatherDimensionNumbers(
        offset_dims=(), collapsed_slice_dims=(0,), start_index_map=(0,))
    return lax.gather(x, perm[:, None], dnums, (1,),
                      mode=lax.GatherScatterMode.PROMISE_IN_BOUNDS)


def _rope_cache():
    pos = jnp.arange(CONTEXT + 1, dtype=jnp.float32)
    inv_freq = 1.0 / (10000.0 ** (
        jnp.arange(0, LATENT_DIM, 2, dtype=jnp.float32) / LATENT_DIM))
    freqs = pos[:, None] * inv_freq[None, :]
    emb = jnp.repeat(freqs, 2, axis=-1)
    alt = jnp.where(jnp.arange(LATENT_DIM) % 2 == 0, -1.0, 1.0).astype(jnp.float32)
    return jnp.cos(emb), jnp.sin(emb) * alt


ROW_SZ = POS_PER_W * LATENT_DIM


def _rope_block(rows_v, flat_v, buf, cos_v, ss_v):
    def r_body(r, carry):
        cs = [cos_v[r, pl.ds(16 * j, 16)] for j in range(4)]
        sg = [ss_v[r, pl.ds(16 * j, 16)] for j in range(4)]

        def b_body(bl, inner):
            for j in range(4):
                x = rows_v[buf, bl, r, pl.ds(16 * j, 16)]
                flat_v[buf, pl.ds(bl * ROW_SZ + r * LATENT_DIM + 16 * j, 16)] = (
                    x * cs[j] + _lane_swap(x) * sg[j])
            return inner

        lax.fori_loop(0, NB, b_body, carry, unroll=2)
        return carry

    lax.fori_loop(0, POS_PER_W, r_body, 0)


def _make_body(b_base, nbatch):
    n_blk = nbatch // NB

    def _body(ctx_hbm, ltab_hbm, cos_hbm, ss_hbm, out_lat,
              cos_v, ss_v, tok_v, rows_v, flat_v, gsem, ssem):
        wid = lax.axis_index("s") * NC + lax.axis_index("c")
        pos0 = wid * POS_PER_W

        pltpu.sync_copy(cos_hbm.at[pl.ds(pos0, POS_PER_W)], cos_v)
        pltpu.sync_copy(ss_hbm.at[pl.ds(pos0, POS_PER_W)], ss_v)
        tok_copies = [
            pltpu.async_copy(
                ctx_hbm.at[pl.ds((b_base + b) * CONTEXT + pos0, POS_PER_W)],
                tok_v.at[b], gsem)
            for b in range(nbatch)
        ]
        for c in tok_copies:
            c.wait()

        def gather_block(blk, buf):
            return [
                pltpu.async_copy(ltab_hbm.at[tok_v.at[blk * NB + bl]],
                                 rows_v.at[buf].at[bl], gsem)
                for bl in range(NB)
            ]

        def scatter_block(blk, buf):
            return [
                pltpu.async_copy(
                    flat_v.at[buf].at[pl.ds(bl * ROW_SZ, ROW_SZ)],
                    out_lat.at[pl.ds(
                        (b_base + blk * NB + bl) * CONTEXT * LATENT_DIM
                        + pos0 * LATENT_DIM, ROW_SZ)],
                    ssem)
                for bl in range(NB)
            ]

        g_prev = gather_block(0, 0)
        s_prev = None
        for blk in range(n_blk):
            buf = blk % NBUF
            for c in g_prev:
                c.wait()
            if blk + 1 < n_blk:
                if s_prev is not None:
                    for c in s_prev:
                        c.wait()
                g_prev = gather_block(blk + 1, (blk + 1) % NBUF)
            _rope_block(rows_v, flat_v, buf, cos_v, ss_v)
            if s_prev is not None and blk + 1 >= n_blk:
                for c in s_prev:
                    c.wait()
            s_prev = scatter_block(blk, buf)
        for c in s_prev:
            c.wait()

    return _body


def _q_body(nt_sref, qtab_ref, cos_ref, ss_ref, out_ref, panels, xs, sem):
    copies = [
        pltpu.make_async_copy(
            qtab_ref.at[pl.ds((nt_sref[i] // 8) * 8, 8), :],
            panels.at[i], sem)
        for i in range(BATCH)
    ]
    for c in copies:
        c.start()
    for c in copies:
        c.wait()
    for i in range(BATCH):
        xs[pl.ds(i, 1), :] = panels[i, pl.ds(nt_sref[i] % 8, 1), :]
    x = xs[...]
    dim = Q_HEADS * LATENT_DIM
    r_i = lax.broadcasted_iota(jnp.int32, (dim, dim), 0)
    c_i = lax.broadcasted_iota(jnp.int32, (dim, dim), 1)
    pss = jnp.where(r_i == (c_i ^ 1), jnp.broadcast_to(ss_ref[...], (dim, dim)),
                    0.0)
    out_ref[...] = x * cos_ref[...] + jnp.dot(
        x, pss, preferred_element_type=jnp.float32)


@jax.jit
def kernel(context_tokens, next_tokens, q_table, latent_table):
    cos_t, ss_t = _rope_cache()
    cos_ctx = cos_t[:CONTEXT]
    ss_ctx = ss_t[:CONTEXT]
    dim = Q_HEADS * LATENT_DIM
    cos_q = jnp.tile(cos_t[CONTEXT], Q_HEADS)[None, :]
    ss_q = jnp.tile(ss_t[CONTEXT], Q_HEADS)[None, :]

    mesh = plsc.VectorSubcoreMesh(core_axis_name="c", subcore_axis_name="s")
    run = functools.partial(
        pl.kernel,
        mesh=mesh,
        compiler_params=pltpu.CompilerParams(use_tc_tiling_on_sc=False),
        out_type=[
            jax.ShapeDtypeStruct((BATCH * CONTEXT * LATENT_DIM,), jnp.float32),
        ],
        scratch_types=[
            pltpu.VMEM((POS_PER_W, LATENT_DIM), jnp.float32),
            pltpu.VMEM((POS_PER_W, LATENT_DIM), jnp.float32),
            pltpu.VMEM((BATCH, POS_PER_W), jnp.int32),
            pltpu.VMEM((NBUF, NB, POS_PER_W, LATENT_DIM), jnp.float32),
            pltpu.VMEM((NBUF, NB * POS_PER_W * LATENT_DIM), jnp.float32),
            pltpu.SemaphoreType.DMA,
            pltpu.SemaphoreType.DMA,
        ],
    )(_make_body(0, BATCH))
    (out_flat,) = run(
        context_tokens.astype(jnp.int32).reshape(-1),
        latent_table, cos_ctx, ss_ctx)
    out_lat = out_flat.reshape(BATCH, CONTEXT, LATENT_DIM)

    out_q = pl.pallas_call(
        _q_body,
        grid_spec=pltpu.PrefetchScalarGridSpec(
            num_scalar_prefetch=1,
            grid=(1,),
            in_specs=[
                pl.BlockSpec(memory_space=pl.ANY),
                pl.BlockSpec((1, dim), lambda i, nt: (0, 0)),
                pl.BlockSpec((1, dim), lambda i, nt: (0, 0)),
            ],
            out_specs=pl.BlockSpec((BATCH, dim), lambda i, nt: (0, 0)),
            scratch_shapes=[
                pltpu.VMEM((BATCH, 8, dim), jnp.float32),
                pltpu.VMEM((BATCH, dim), jnp.float32),
                pltpu.SemaphoreType.DMA,
            ],
        ),
        out_shape=jax.ShapeDtypeStruct((BATCH, dim), jnp.float32),
    )(next_tokens.astype(jnp.int32), q_table, cos_q, ss_q)

    q = out_q.reshape(BATCH, Q_HEADS, 1, LATENT_DIM)
    return (q, out_lat)
